# baseline (speedup 1.0000x reference)
import functools

import jax
import jax.numpy as jnp
import numpy as np
from jax.experimental import pallas as pl
from jax.experimental.pallas import tpu as pltpu

K = 1024
CD = 10
D = 64
SCALE = 1.0
INV_TEMP = 100.0
ENT_W = 0.1
COMMIT_W = 0.25
GAMMA = 1.0
B, N = 8, 4096
TOKENS = B * N
TILE = 512
GRID = TOKENS // TILE

_s = 2.0 * INV_TEMP * SCALE
_CT73 = np.zeros((CD, 136), dtype=np.float32)
for _d in range(7):
    _J = np.arange(128)
    _CT73[_d, :128] = _s * (2.0 * ((_J >> (6 - _d)) & 1) - 1.0)
for _d in range(7, CD):
    _L = np.arange(8)
    _CT73[_d, 128:136] = _s * (2.0 * ((_L >> (9 - _d)) & 1) - 1.0)

_MCOLS = np.zeros((CD, 2), dtype=np.float32)
_MCOLS[:7, 0] = _s
_MCOLS[7:, 1] = _s

_IMASK = (2 ** np.arange(CD - 1, -1, -1)).astype(np.float32)


def _lfq_body(z_ref, wi_ref, bi_ref, wo_ref, bo_ref, ct_ref, mc_ref,
              out_ref, idx_ref, aux_ref,
              avg_acc, sums_acc):
    step = pl.program_id(0)

    @pl.when(step == 0)
    def _init():
        avg_acc[...] = jnp.zeros_like(avg_acc)
        sums_acc[0] = 0.0
        sums_acc[1] = 0.0

    z = z_ref[...]
    x = jax.lax.dot_general(z, wi_ref[...], (((1,), (1,)), ((), ())),
                            preferred_element_type=jnp.float32) + bi_ref[...]
    pos = x > 0
    q = jnp.where(pos, SCALE, -SCALE).astype(jnp.float32)

    y2 = jax.lax.dot_general(q, wo_ref[...], (((1,), (1,)), ((), ())),
                             preferred_element_type=jnp.float32)
    out_ref[...] = y2[:, :D] + bo_ref[...]
    idx_ref[...] = (y2[:, D:D + 1] + (float(K) - 1.0) * 0.5).astype(jnp.int32)

    commit_tile = jnp.sum((x - q) ** 2)

    y = jax.lax.dot_general(x, ct_ref[...], (((1,), (0,)), ((), ())),
                            preferred_element_type=jnp.float32)
    m = jax.lax.dot_general(jnp.abs(x), mc_ref[...], (((1,), (0,)), ((), ())),
                            preferred_element_type=jnp.float32)
    lp7 = y[:, :128] - m[:, 0:1]
    lp3 = y[:, 128:136] - m[:, 1:2]
    e7 = jnp.exp(lp7)
    e3 = jnp.exp(lp3)
    ones1 = jnp.ones((128, 1), dtype=jnp.float32)
    s7 = jax.lax.dot_general(e7, ones1, (((1,), (0,)), ((), ())),
                             preferred_element_type=jnp.float32)
    w7 = jax.lax.dot_general(e7 * lp7, ones1, (((1,), (0,)), ((), ())),
                             preferred_element_type=jnp.float32)
    s3 = jnp.sum(e3, axis=1, keepdims=True)
    w3 = jnp.sum(e3 * lp3, axis=1, keepdims=True)
    r7 = 1.0 / s7
    r3 = 1.0 / s3
    h = jnp.log(s7 * s3) - w7 * r7 - w3 * r3
    ent_tile = jnp.sum(h)

    p3s = e3 * (r7 * r3)
    avg_acc[...] += jax.lax.dot_general(e7, p3s, (((0,), (0,)), ((), ())),
                                        preferred_element_type=jnp.float32)

    sums_acc[0] += ent_tile
    sums_acc[1] += commit_tile

    @pl.when(step == GRID - 1)
    def _fin():
        nt = float(TOKENS)
        pse = sums_acc[0] / nt
        ap = avg_acc[...] / nt
        ce = jnp.sum(-ap * jnp.log(jnp.clip(ap, 1e-20, None)))
        commit = sums_acc[1] / (nt * CD)
        aux = (pse - GAMMA * ce) * ENT_W + COMMIT_W * commit
        aux_ref[...] = jnp.reshape(aux, (1, 1))


@functools.partial(jax.jit, static_argnames=())
def kernel(z_e_x, W_in, b_in, W_out, b_out):
    z2 = z_e_x.reshape(TOKENS, D)
    bi = b_in.reshape(1, CD)
    bo = b_out.reshape(1, D)
    ct = jnp.asarray(_CT73)
    mc = jnp.asarray(_MCOLS)
    wo_aug = jnp.concatenate([W_out, jnp.asarray(_IMASK)[None, :] * 0.5], axis=0)

    out2, idx2, aux = pl.pallas_call(
        _lfq_body,
        grid=(GRID,),
        in_specs=[
            pl.BlockSpec((TILE, D), lambda i: (i, 0)),
            pl.BlockSpec((CD, D), lambda i: (0, 0)),
            pl.BlockSpec((1, CD), lambda i: (0, 0)),
            pl.BlockSpec((D + 1, CD), lambda i: (0, 0)),
            pl.BlockSpec((1, D), lambda i: (0, 0)),
            pl.BlockSpec((CD, 136), lambda i: (0, 0)),
            pl.BlockSpec((CD, 2), lambda i: (0, 0)),
        ],
        out_specs=[
            pl.BlockSpec((TILE, D), lambda i: (i, 0)),
            pl.BlockSpec((TILE, 1), lambda i: (i, 0)),
            pl.BlockSpec((1, 1), lambda i: (0, 0)),
        ],
        out_shape=[
            jax.ShapeDtypeStruct((TOKENS, D), jnp.float32),
            jax.ShapeDtypeStruct((TOKENS, 1), jnp.int32),
            jax.ShapeDtypeStruct((1, 1), jnp.float32),
        ],
        scratch_shapes=[
            pltpu.VMEM((128, 8), jnp.float32),
            pltpu.SMEM((2,), jnp.float32),
        ],
    )(z2, W_in, bi, wo_aug, bo, ct, mc)

    out = out2.reshape(B, N, D)
    indices = idx2.reshape(B, N)
    aux_loss = aux.reshape(())
    return (out, indices, aux_loss)
